# baseline (speedup 1.0000x reference)
import functools

import jax
import jax.numpy as jnp
from jax import lax
from jax.experimental import pallas as pl
from jax.experimental.pallas import tpu as pltpu
from jax.experimental.pallas import tpu_sc as plsc

VOCAB = 1000
N_EMBD = 32
T = 8
ROWS = 4096 * 8
NW = 32
ROWS_PER_W = ROWS // NW
CHUNK = 128
NCHUNK = ROWS_PER_W // CHUNK
BLK = 2048
GRID = ROWS // BLK


def _sc_gather_kernel(table_hbm, idx_hbm, out_hbm, idx_v, rows_v, sem):
    wid = lax.axis_index("s") * 2 + lax.axis_index("c")
    base = wid * NCHUNK
    pltpu.sync_copy(idx_hbm.at[pl.ds(base, NCHUNK)], idx_v)
    for j in range(NCHUNK):
        pltpu.async_copy(table_hbm.at[idx_v.at[j]], rows_v.at[j], sem).wait()
        pltpu.sync_copy(
            rows_v.at[j],
            out_hbm.at[pl.ds(wid * ROWS_PER_W + j * CHUNK, CHUNK)],
        )


@jax.jit
def _sc_gather(tok_table, idx2):
    mesh = plsc.VectorSubcoreMesh(core_axis_name="c", subcore_axis_name="s")
    return pl.kernel(
        _sc_gather_kernel,
        mesh=mesh,
        out_type=jax.ShapeDtypeStruct((ROWS, N_EMBD), jnp.float32),
        scratch_types=[
            pltpu.VMEM((NCHUNK, CHUNK), jnp.int32),
            pltpu.VMEM((NCHUNK, CHUNK, N_EMBD), jnp.float32),
            pltpu.SemaphoreType.DMA,
        ],
        compiler_params=pltpu.CompilerParams(use_tc_tiling_on_sc=False),
    )(tok_table, idx2)


def _tc_head_kernel(x_ref, pos_ref, w_ref, b_ref, t_ref, logits_ref, loss_ref,
                    buf, sem):
    i = pl.program_id(0)
    x = x_ref[...]
    xp = x.reshape(BLK // T, T, N_EMBD) + pos_ref[...][None, :, :]
    xp = xp.reshape(BLK, N_EMBD)
    logits = (
        jnp.dot(xp, w_ref[...], preferred_element_type=jnp.float32,
                precision=lax.Precision.DEFAULT)
        + b_ref[...]
    )
    b2 = i % 2

    @pl.when(i >= 2)
    def _drain():
        pltpu.make_async_copy(
            buf.at[b2], logits_ref.at[pl.ds(i * BLK, BLK)], sem.at[b2]
        ).wait()

    buf[b2] = logits
    pltpu.make_async_copy(
        buf.at[b2], logits_ref.at[pl.ds(i * BLK, BLK)], sem.at[b2]
    ).start()

    rowmax = jnp.max(logits, axis=1, keepdims=True)
    se = jnp.sum(jnp.exp(logits - rowmax), axis=1)
    viota = lax.broadcasted_iota(jnp.int32, (BLK, VOCAB), 1)
    tmask = viota == t_ref[...]
    tlogit = jnp.sum(jnp.where(tmask, logits, 0.0), axis=1)
    bs = jnp.sum(jnp.log(se) + rowmax[:, 0] - tlogit).reshape(1, 1)

    @pl.when(i == 0)
    def _init():
        loss_ref2 = loss_ref
        loss_ref2[...] = jnp.zeros((1, 1), jnp.float32)

    loss_ref[...] += bs

    @pl.when(i == pl.num_programs(0) - 1)
    def _fin():
        loss_ref[...] = loss_ref[...] / ROWS
        pltpu.make_async_copy(
            buf.at[1 - b2], logits_ref.at[pl.ds(i * BLK, BLK)], sem.at[1 - b2]
        ).wait()
        pltpu.make_async_copy(
            buf.at[b2], logits_ref.at[pl.ds(i * BLK, BLK)], sem.at[b2]
        ).wait()


@jax.jit
def _tc_head(x, pos_table, W, b2, t2):
    return pl.pallas_call(
        _tc_head_kernel,
        grid=(GRID,),
        in_specs=[
            pl.BlockSpec((BLK, N_EMBD), lambda i: (i, 0)),
            pl.BlockSpec((T, N_EMBD), lambda i: (0, 0)),
            pl.BlockSpec((N_EMBD, VOCAB), lambda i: (0, 0)),
            pl.BlockSpec((1, VOCAB), lambda i: (0, 0)),
            pl.BlockSpec((BLK, 1), lambda i: (i, 0)),
        ],
        out_specs=[
            pl.BlockSpec(memory_space=pltpu.HBM),
            pl.BlockSpec((1, 1), lambda i: (0, 0)),
        ],
        out_shape=[
            jax.ShapeDtypeStruct((ROWS, VOCAB), jnp.float32),
            jax.ShapeDtypeStruct((1, 1), jnp.float32),
        ],
        scratch_shapes=[
            pltpu.VMEM((2, BLK, VOCAB), jnp.float32),
            pltpu.SemaphoreType.DMA((2,)),
        ],
    )(x, pos_table, W, b2, t2)


def kernel(idx, targets, tok_table, pos_table, W, b):
    idx2 = idx.reshape(NW * NCHUNK, CHUNK).astype(jnp.int32)
    x = _sc_gather(tok_table, idx2)
    t2 = targets.reshape(ROWS, 1).astype(jnp.int32)
    logits2, loss = _tc_head(x, pos_table, W, b.reshape(1, VOCAB), t2)
    return (logits2, loss[0, 0])
